# Initial kernel scaffold; baseline (speedup 1.0000x reference)
import jax
import jax.numpy as jnp
from jax.experimental import pallas as pl


def kernel(x_num, x_cat, emb_tables, W1, b1, W2, b2, W3, b3):
    raise NotImplementedError("write your pallas kernel here")


# speedup vs baseline: 7.1495x; 7.1495x over previous
import functools

import jax
import jax.numpy as jnp
from jax import lax
from jax.experimental import pallas as pl
from jax.experimental.pallas import tpu as pltpu
from jax.experimental.pallas import tpu_sc as plsc

B = 16384
F = 26
V = 100000
D = 16
NUM = 13
H1 = 128
H2 = 64
N = B * F


_CHUNK = 128


def _make_sc_gather():
    info = plsc.get_sparse_core_info()
    nc, ns = info.num_cores, info.num_subcores
    nw = nc * ns
    per_w = N // nw
    n_chunks = per_w // _CHUNK
    mesh = plsc.VectorSubcoreMesh(core_axis_name="c", subcore_axis_name="s")

    @functools.partial(
        pl.kernel,
        mesh=mesh,
        out_type=jax.ShapeDtypeStruct((N, D), jnp.float32),
        scratch_types=[
            pltpu.VMEM((_CHUNK,), jnp.int32),
            pltpu.VMEM((_CHUNK, D), jnp.float32),
            pltpu.SemaphoreType.DMA,
        ],
        compiler_params=pltpu.CompilerParams(use_tc_tiling_on_sc=False),
    )
    def gather_k(table_hbm, idx_hbm, out_hbm, idx_v, rows_v, sem):
        wid = lax.axis_index("s") * nc + lax.axis_index("c")
        base_w = wid * per_w

        def body(j, carry):
            base = base_w + j * _CHUNK
            pltpu.sync_copy(idx_hbm.at[pl.ds(base, _CHUNK)], idx_v)
            pltpu.async_copy(table_hbm.at[idx_v], rows_v, sem).wait()
            pltpu.sync_copy(rows_v, out_hbm.at[pl.ds(base, _CHUNK)])
            return carry

        lax.fori_loop(0, n_chunks, body, 0)

    return gather_k


_sc_gather = _make_sc_gather()


_BB = 1024


def _mlp_body(emb_ref, xn_ref, w1a_ref, w1b_ref, b1_ref, w2_ref, b2_ref,
              w3_ref, b3_ref, o_ref):
    h = jnp.dot(emb_ref[...], w1a_ref[...], preferred_element_type=jnp.float32)
    h = h + jnp.dot(xn_ref[...], w1b_ref[...], preferred_element_type=jnp.float32)
    h = jnp.maximum(h + b1_ref[...], 0.0)
    h = jnp.maximum(
        jnp.dot(h, w2_ref[...], preferred_element_type=jnp.float32) + b2_ref[...],
        0.0)
    o_ref[...] = (
        jnp.dot(h, w3_ref[...], preferred_element_type=jnp.float32) + b3_ref[...])


def _mlp(emb, x_num, w1a, w1b, b1, w2, b2, w3, b3):
    grid = (B // _BB,)
    full = lambda shape: pl.BlockSpec(shape, lambda i: (0, 0))
    return pl.pallas_call(
        _mlp_body,
        grid=grid,
        in_specs=[
            pl.BlockSpec((_BB, F * D), lambda i: (i, 0)),
            pl.BlockSpec((_BB, NUM), lambda i: (i, 0)),
            full(w1a.shape),
            full(w1b.shape),
            full((1, H1)),
            full(w2.shape),
            full((1, H2)),
            full(w3.shape),
            full((1, 1)),
        ],
        out_specs=pl.BlockSpec((_BB, 1), lambda i: (i, 0)),
        out_shape=jax.ShapeDtypeStruct((B, 1), jnp.float32),
    )(emb, x_num, w1a, w1b, b1.reshape(1, H1), w2, b2.reshape(1, H2), w3,
      b3.reshape(1, 1))


def kernel(x_num, x_cat, emb_tables, W1, b1, W2, b2, W3, b3):
    table_flat = emb_tables.reshape(F * V, D)
    idx_flat = (x_cat.astype(jnp.int32)
                + (jnp.arange(F, dtype=jnp.int32) * V)[None, :]).reshape(N)
    gathered = _sc_gather(table_flat, idx_flat)
    emb = gathered.reshape(B, F * D)
    w1a = W1[:F * D]
    w1b = W1[F * D:]
    return _mlp(emb, x_num, w1a, w1b, b1, W2, b2, W3, b3)


# speedup vs baseline: 7.6839x; 1.0748x over previous
import functools

import jax
import jax.numpy as jnp
from jax import lax
from jax.experimental import pallas as pl
from jax.experimental.pallas import tpu as pltpu
from jax.experimental.pallas import tpu_sc as plsc

B = 16384
F = 26
V = 100000
D = 16
NUM = 13
H1 = 128
H2 = 64
N = B * F


_CHUNK = 128


def _make_sc_gather():
    info = plsc.get_sparse_core_info()
    nc, ns = info.num_cores, info.num_subcores
    nw = nc * ns
    per_w = N // nw
    n_chunks = per_w // _CHUNK
    mesh = plsc.VectorSubcoreMesh(core_axis_name="c", subcore_axis_name="s")

    @functools.partial(
        pl.kernel,
        mesh=mesh,
        out_type=jax.ShapeDtypeStruct((N, D), jnp.float32),
        scratch_types=[
            pltpu.VMEM((per_w,), jnp.int32),
            pltpu.VMEM((_CHUNK, D), jnp.float32),
            pltpu.VMEM((_CHUNK, D), jnp.float32),
            pltpu.SemaphoreType.DMA,
            pltpu.SemaphoreType.DMA,
        ],
        compiler_params=pltpu.CompilerParams(use_tc_tiling_on_sc=False),
    )
    def gather_k(table_hbm, idx_hbm, out_hbm, idx_all, rows0, rows1, sem0, sem1):
        wid = lax.axis_index("s") * nc + lax.axis_index("c")
        base_w = wid * per_w
        pltpu.sync_copy(idx_hbm.at[pl.ds(base_w, per_w)], idx_all)
        rows = (rows0, rows1)
        sems = (sem0, sem1)

        def gather_start(j, b):
            pltpu.async_copy(
                table_hbm.at[idx_all.at[pl.ds(j * _CHUNK, _CHUNK)]],
                rows[b], sems[b])

        def gather_wait(b):
            pltpu.make_async_copy(
                table_hbm.at[idx_all.at[pl.ds(0, _CHUNK)]], rows[b],
                sems[b]).wait()

        gather_start(0, 0)

        def body(jj, carry):
            for b in range(2):
                j = jj * 2 + b

                @pl.when(j + 1 < n_chunks)
                def _():
                    gather_start(j + 1, 1 - b)

                gather_wait(b)
                pltpu.sync_copy(rows[b],
                                out_hbm.at[pl.ds(base_w + j * _CHUNK, _CHUNK)])
            return carry

        lax.fori_loop(0, n_chunks // 2, body, 0)

    return gather_k


_sc_gather = _make_sc_gather()


_BB = 1024


def _mlp_body(emb_ref, xn_ref, w1a_ref, w1b_ref, b1_ref, w2_ref, b2_ref,
              w3_ref, b3_ref, o_ref):
    h = jnp.dot(emb_ref[...], w1a_ref[...], preferred_element_type=jnp.float32)
    h = h + jnp.dot(xn_ref[...], w1b_ref[...], preferred_element_type=jnp.float32)
    h = jnp.maximum(h + b1_ref[...], 0.0)
    h = jnp.maximum(
        jnp.dot(h, w2_ref[...], preferred_element_type=jnp.float32) + b2_ref[...],
        0.0)
    o_ref[...] = (
        jnp.dot(h, w3_ref[...], preferred_element_type=jnp.float32) + b3_ref[...])


def _mlp(emb, x_num, w1a, w1b, b1, w2, b2, w3, b3):
    grid = (B // _BB,)
    full = lambda shape: pl.BlockSpec(shape, lambda i: (0, 0))
    return pl.pallas_call(
        _mlp_body,
        grid=grid,
        in_specs=[
            pl.BlockSpec((_BB, F * D), lambda i: (i, 0)),
            pl.BlockSpec((_BB, NUM), lambda i: (i, 0)),
            full(w1a.shape),
            full(w1b.shape),
            full((1, H1)),
            full(w2.shape),
            full((1, H2)),
            full(w3.shape),
            full((1, 1)),
        ],
        out_specs=pl.BlockSpec((_BB, 1), lambda i: (i, 0)),
        out_shape=jax.ShapeDtypeStruct((B, 1), jnp.float32),
    )(emb, x_num, w1a, w1b, b1.reshape(1, H1), w2, b2.reshape(1, H2), w3,
      b3.reshape(1, 1))


def kernel(x_num, x_cat, emb_tables, W1, b1, W2, b2, W3, b3):
    table_flat = emb_tables.reshape(F * V, D)
    idx_flat = (x_cat.astype(jnp.int32)
                + (jnp.arange(F, dtype=jnp.int32) * V)[None, :]).reshape(N)
    gathered = _sc_gather(table_flat, idx_flat)
    emb = gathered.reshape(B, F * D)
    w1a = W1[:F * D]
    w1b = W1[F * D:]
    return _mlp(emb, x_num, w1a, w1b, b1, W2, b2, W3, b3)
